# baseline (speedup 1.0000x reference)
import functools
import math

import jax
import jax.numpy as jnp
from jax import lax
from jax.experimental import pallas as pl
from jax.experimental.pallas import tpu as pltpu
from jax.experimental.pallas import tpu_sc as plsc

_EMB = 64
_SCALE = math.sqrt(_EMB)
_NC = 2
_NS = 16
_NW = _NC * _NS
_C = 128
_LANES = 16


@functools.partial(jax.jit, static_argnames=("n_chunks",))
def _embed(tok, table, n_chunks):
    bpw = n_chunks * _C
    b_total = _NW * bpw

    mesh = plsc.VectorSubcoreMesh(core_axis_name="c", subcore_axis_name="s")

    @functools.partial(
        pl.kernel,
        out_type=jax.ShapeDtypeStruct((b_total, _EMB), jnp.float32),
        mesh=mesh,
        compiler_params=pltpu.CompilerParams(use_tc_tiling_on_sc=False),
        scratch_types=[
            pltpu.VMEM((n_chunks, _C), jnp.int32),
            pltpu.VMEM((_C, _EMB), jnp.float32),
            pltpu.VMEM((_C, _EMB), jnp.float32),
            pltpu.SemaphoreType.DMA,
            pltpu.SemaphoreType.DMA,
            pltpu.SemaphoreType.DMA,
            pltpu.SemaphoreType.DMA,
        ],
    )
    def k(tok_hbm, table_hbm, out_hbm, idx_v, rows0, rows1, g0, g1, s0, s1):
        wid = lax.axis_index("s") * _NC + lax.axis_index("c")
        base = wid * bpw
        pltpu.sync_copy(tok_hbm.at[wid], idx_v)

        bufs = ((rows0, g0, s0), (rows1, g1, s1))

        pltpu.async_copy(table_hbm.at[idx_v.at[0]], rows0, g0)

        def pair(g, _):
            for b in range(2):
                i = g * 2 + b
                rows_b, gs_b, ss_b = bufs[b]
                rows_n, gs_n, ss_n = bufs[1 - b]

                @pl.when(i > 0)
                def _():
                    pltpu.make_async_copy(
                        rows_n, out_hbm.at[pl.ds(base + (i - 1) * _C, _C)], ss_n
                    ).wait()

                @pl.when(i + 1 < n_chunks)
                def _():
                    pltpu.async_copy(table_hbm.at[idx_v.at[i + 1]], rows_n, gs_n)

                pltpu.make_async_copy(
                    table_hbm.at[idx_v.at[i]], rows_b, gs_b
                ).wait()

                @plsc.parallel_loop(0, _C, step=1, unroll=8)
                def _(j):
                    for l in range(_EMB // _LANES):
                        sl = (j, pl.ds(l * _LANES, _LANES))
                        rows_b[sl] = rows_b[sl] * _SCALE

                pltpu.async_copy(
                    rows_b, out_hbm.at[pl.ds(base + i * _C, _C)], ss_b
                )
            return 0

        lax.fori_loop(0, n_chunks // 2, pair, 0)

        last = n_chunks - 1
        pltpu.make_async_copy(
            rows1, out_hbm.at[pl.ds(base + last * _C, _C)], s1
        ).wait()

    return k(tok, table)


def kernel(tokens, table):
    b0, b1 = tokens.shape
    b_total = b0 * b1
    n_chunks = b_total // (_NW * _C)
    tok = tokens.astype(jnp.int32).reshape(_NW, n_chunks, _C)
    out = _embed(tok, table, n_chunks)
    return out.reshape(b0, b1, _EMB)


# speedup vs baseline: 1.0352x; 1.0352x over previous
import functools
import math

import jax
import jax.numpy as jnp
from jax import lax
from jax.experimental import pallas as pl
from jax.experimental.pallas import tpu as pltpu
from jax.experimental.pallas import tpu_sc as plsc

_EMB = 64
_SCALE = math.sqrt(_EMB)
_NC = 2
_NS = 16
_NW = _NC * _NS
_C = 128
_G = 4
_CC = _C * _G
_LANES = 16


@functools.partial(jax.jit, static_argnames=("n_steps",))
def _embed(tok, table, n_steps):
    bpw = n_steps * _CC
    b_total = _NW * bpw

    mesh = plsc.VectorSubcoreMesh(core_axis_name="c", subcore_axis_name="s")

    @functools.partial(
        pl.kernel,
        out_type=jax.ShapeDtypeStruct((b_total, _EMB), jnp.float32),
        mesh=mesh,
        compiler_params=pltpu.CompilerParams(use_tc_tiling_on_sc=False),
        scratch_types=[
            pltpu.VMEM((n_steps * _G, _C), jnp.int32),
            pltpu.VMEM((_CC, _EMB), jnp.float32),
            pltpu.VMEM((_CC, _EMB), jnp.float32),
            pltpu.SemaphoreType.DMA,
            pltpu.SemaphoreType.DMA,
            pltpu.SemaphoreType.DMA,
            pltpu.SemaphoreType.DMA,
        ],
    )
    def k(tok_hbm, table_hbm, out_hbm, idx_v, rows0, rows1, g0, g1, s0, s1):
        wid = lax.axis_index("s") * _NC + lax.axis_index("c")
        base = wid * bpw
        pltpu.sync_copy(tok_hbm.at[wid], idx_v)

        bufs = ((rows0, g0, s0), (rows1, g1, s1))

        def gstart(i, rows, gsem):
            for j in range(_G):
                pltpu.async_copy(
                    table_hbm.at[idx_v.at[i * _G + j]],
                    rows.at[pl.ds(j * _C, _C)],
                    gsem,
                )

        def gwait(i, rows, gsem):
            for j in range(_G):
                pltpu.make_async_copy(
                    table_hbm.at[idx_v.at[i * _G + j]],
                    rows.at[pl.ds(j * _C, _C)],
                    gsem,
                ).wait()

        gstart(0, rows0, g0)

        def pair(g, _):
            for b in range(2):
                i = g * 2 + b
                rows_b, gs_b, ss_b = bufs[b]
                rows_n, gs_n, ss_n = bufs[1 - b]

                @pl.when(i > 0)
                def _():
                    pltpu.make_async_copy(
                        rows_n,
                        out_hbm.at[pl.ds(base + (i - 1) * _CC, _CC)],
                        ss_n,
                    ).wait()

                @pl.when(i + 1 < n_steps)
                def _():
                    gstart(i + 1, rows_n, gs_n)

                gwait(i, rows_b, gs_b)

                @plsc.parallel_loop(0, _CC, step=1, unroll=8)
                def _(j):
                    for l in range(_EMB // _LANES):
                        sl = (j, pl.ds(l * _LANES, _LANES))
                        rows_b[sl] = rows_b[sl] * _SCALE

                pltpu.async_copy(
                    rows_b, out_hbm.at[pl.ds(base + i * _CC, _CC)], ss_b
                )
            return 0

        lax.fori_loop(0, n_steps // 2, pair, 0)

        last = n_steps - 1
        pltpu.make_async_copy(
            rows1, out_hbm.at[pl.ds(base + last * _CC, _CC)], s1
        ).wait()

    return k(tok, table)


def kernel(tokens, table):
    b0, b1 = tokens.shape
    b_total = b0 * b1
    n_steps = b_total // (_NW * _CC)
    tok = tokens.astype(jnp.int32).reshape(_NW, n_steps * _G, _C)
    out = _embed(tok, table, n_steps)
    return out.reshape(b0, b1, _EMB)
